# baseline (speedup 1.0000x reference)
import jax
import jax.numpy as jnp
from jax.experimental import pallas as pl
from jax.experimental.pallas import tpu as pltpu
from jax.experimental.pallas import tpu_sc as plsc

_N_E = 8192
_D = 256
_N_TOK = 4608
_BM = 512
_BN = 1024
_NI = _N_TOK // _BM
_NJ = _N_E // _BN
_GW = 128
_EPS = 1e-12


def _argmin_body(z_ref, en_ref, esq_ref, idx_ref, loss_ref, enb_s):
    i = pl.program_id(0)

    @pl.when(i == 0)
    def _prep_codebook():
        for j in range(_NJ):
            ent = jnp.transpose(en_ref[j * _BN:(j + 1) * _BN, :],
                                (1, 0))
            enb_s[j] = ent.astype(jnp.bfloat16)

    zn = z_ref[...]
    zsq = jnp.sum(zn * zn, keepdims=True)
    znm2 = (-2.0 * zn).astype(jnp.bfloat16)

    nr = _BM // 128
    rv = [None] * nr
    ri = [None] * nr
    for j in range(_NJ):
        d2 = jax.lax.dot_general(
            znm2, enb_s[j],
            dimension_numbers=(((1,), (0,)), ((), ())),
            preferred_element_type=jnp.float32)
        score = d2 + esq_ref[:, j * _BN:(j + 1) * _BN]
        for r in range(nr):
            rvc, ric = rv[r], ri[r]
            for g in range(_BN // 128):
                sg = score[r * 128:(r + 1) * 128, g * 128:(g + 1) * 128]
                fold = j * (_BN // 128) + g
                if rvc is None:
                    rvc = sg
                    ric = jnp.zeros((128, 128), jnp.int32)
                else:
                    m = sg < rvc
                    rvc = jnp.where(m, sg, rvc)
                    ric = jnp.where(m, jnp.full((128, 128), fold,
                                                jnp.int32), ric)
            rv[r], ri[r] = rvc, ric

    rva = jnp.concatenate(rv, axis=0)
    ria = jnp.concatenate(ri, axis=0)
    rvt = rva.T
    rit = ria.T
    sub = jax.lax.broadcasted_iota(jnp.int32, (128, _BM), 0)
    gidx = rit * 128 + sub
    mv = jnp.min(rvt, axis=0, keepdims=True)
    cand = jnp.where(rvt == mv, gidx, jnp.full((128, _BM), 2**31 - 1,
                                               jnp.int32))
    idx_ref[...] = jnp.min(cand, axis=0, keepdims=True).reshape(1, 1, _BM)
    part = zsq + jnp.sum(mv, keepdims=True)
    prev = jnp.where(i == 0, jnp.zeros((1, 1), jnp.float32),
                     loss_ref[...])
    tot = prev + part
    scale = 1.25 / float(_N_TOK * _D)
    tot = jnp.where(i == _NI - 1, tot * scale, tot)
    loss_ref[...] = tot


def _argmin_call(zn, en, esq):
    return pl.pallas_call(
        _argmin_body,
        grid=(_NI,),
        in_specs=[
            pl.BlockSpec((_BM, _D), lambda i: (i, 0)),
            pl.BlockSpec((_N_E, _D), lambda i: (0, 0)),
            pl.BlockSpec((1, _N_E), lambda i: (0, 0)),
        ],
        out_specs=[
            pl.BlockSpec((1, 1, _BM), lambda i: (i, 0, 0)),
            pl.BlockSpec((1, 1), lambda i: (0, 0)),
        ],
        out_shape=[
            jax.ShapeDtypeStruct((_NI, 1, _BM), jnp.int32),
            jax.ShapeDtypeStruct((1, 1), jnp.float32),
        ],
        scratch_shapes=[
            pltpu.VMEM((_NJ, _D, _BN), jnp.bfloat16),
        ],
        compiler_params=pltpu.CompilerParams(
            dimension_semantics=("arbitrary",)),
    )(zn, en, esq)


def _gather_rows(table, idx_row):
    mesh = plsc.VectorSubcoreMesh(core_axis_name="core",
                                  subcore_axis_name="subcore")

    @pl.kernel(out_type=jax.ShapeDtypeStruct((_N_TOK, _D), table.dtype),
               mesh=mesh)
    def k(x_hbm, i_hbm, o_hbm):
        def body(i_vmem, o_vmem):
            pltpu.sync_copy(x_hbm.at[i_vmem.at[0]], o_vmem)

        pltpu.emit_pipeline(
            body,
            grid=(_N_TOK // _GW,),
            in_specs=[pl.BlockSpec((1, _GW), index_map=lambda i: (0, i))],
            out_specs=[pl.BlockSpec((_GW, _D), index_map=lambda i: (i, 0))],
            core_axis_name=("core", "subcore"),
            dimension_semantics=(pltpu.PARALLEL,),
        )(i_hbm, o_hbm)

    return k(table, idx_row)


def _norm(x):
    n = jnp.linalg.norm(x, axis=-1, keepdims=True)
    return x / jnp.maximum(n, _EPS)


def kernel(z, embedding_weight):
    z_flat = z.reshape(_N_TOK, _D)
    zn = _norm(z_flat)
    en = _norm(embedding_weight)
    esq = jnp.sum(en ** 2, axis=1).reshape(1, _N_E)
    idx2, loss = _argmin_call(zn, en, esq)
    idx = idx2.reshape(_N_TOK)
    z_qnorm = _gather_rows(en, idx2.reshape(1, _N_TOK))
    return (z_qnorm.reshape(z.shape), loss.reshape(()), idx)
